# Initial kernel scaffold; baseline (speedup 1.0000x reference)
import functools

import jax
import jax.numpy as jnp
from jax import lax
from jax.experimental import pallas as pl
from jax.experimental.pallas import tpu as pltpu
from jax.experimental.pallas import tpu_sc as plsc

N_PAD = 10240
ROWS_PT = N_PAD // 16
NW = 32


def _make_spmm(d, n_blk, with_cnt):
    mesh = plsc.VectorSubcoreMesh(core_axis_name="c", subcore_axis_name="s")

    out_type = [jax.ShapeDtypeStruct((2, N_PAD, d), jnp.float32)]
    scratch = [
        pltpu.VMEM((n_blk, 128), jnp.int32),
        pltpu.VMEM((n_blk, 128), jnp.int32),
        pltpu.VMEM((128, d), jnp.float32),
        pltpu.VMEM_SHARED((N_PAD, d), jnp.float32),
        pltpu.SemaphoreType.DMA,
    ]
    if with_cnt:
        out_type.append(jax.ShapeDtypeStruct((2, N_PAD, 16), jnp.float32))
        scratch += [
            pltpu.VMEM((128, 16), jnp.float32),
            pltpu.VMEM_SHARED((N_PAD, 16), jnp.float32),
        ]

    def body(x_hbm, src_hbm, dst_hbm, zeros_hbm, *rest):
        if with_cnt:
            (zeros16_hbm, ones_hbm, out_hbm, cnt_hbm,
             sidx, didx, rows, acc, sem, ones_v, acc_cnt) = rest
        else:
            out_hbm, sidx, didx, rows, acc, sem = rest
        c = lax.axis_index("c")
        s = lax.axis_index("s")
        w = c * 16 + s
        r0 = s * ROWS_PT

        pltpu.sync_copy(zeros_hbm, acc.at[pl.ds(r0, ROWS_PT)])
        pltpu.sync_copy(src_hbm.at[w], sidx)
        pltpu.sync_copy(dst_hbm.at[w], didx)
        if with_cnt:
            pltpu.sync_copy(zeros16_hbm, acc_cnt.at[pl.ds(r0, ROWS_PT)])
            pltpu.sync_copy(ones_hbm, ones_v)
        plsc.subcore_barrier()

        def step(j, carry):
            pltpu.async_copy(x_hbm.at[sidx.at[j]], rows, sem).wait()
            pltpu.sync_copy(rows, acc.at[didx.at[j]], add=True)
            if with_cnt:
                pltpu.sync_copy(ones_v, acc_cnt.at[didx.at[j]], add=True)
            return carry

        lax.fori_loop(0, n_blk, step, 0)

        plsc.subcore_barrier()
        pltpu.sync_copy(acc.at[pl.ds(r0, ROWS_PT)],
                        out_hbm.at[c, pl.ds(r0, ROWS_PT)])
        if with_cnt:
            pltpu.sync_copy(acc_cnt.at[pl.ds(r0, ROWS_PT)],
                            cnt_hbm.at[c, pl.ds(r0, ROWS_PT)])

    return pl.kernel(body, out_type=out_type, mesh=mesh,
                     scratch_types=scratch)


def _dot_t(a, w):
    return lax.dot_general(a, w, (((1,), (1,)), ((), ())),
                           preferred_element_type=jnp.float32)


_R = 256
_GRID = N_PAD // _R


def _full(shape):
    return pl.BlockSpec(shape, lambda i: (0,) * len(shape))


def _rows(minor):
    return pl.BlockSpec((_R, minor), lambda i: (i, 0))


def _pair(minor):
    return pl.BlockSpec((2, _R, minor), lambda i: (0, i, 0))


def _dense1_body(s1, c1, x, w1l, b1l, w1r, w2, t2_o, dinv_o, cntc_o):
    cnt = c1[0][:, :1] + c1[1][:, :1]
    cntc = jnp.maximum(cnt, 1.0)
    agg = (s1[0] + s1[1]) / cntc
    h1 = jnp.maximum(
        _dot_t(agg, w1l[...]) + b1l[...] + _dot_t(x[...], w1r[...]), 0.0)
    dinv = lax.rsqrt(cnt + 1.0)
    t2_o[...] = dinv * _dot_t(h1, w2[...])
    dinv_o[...] = jnp.broadcast_to(dinv, (_R, 16))
    cntc_o[...] = jnp.broadcast_to(cntc, (_R, 16))


def _dense2_body(s2, t2, dinv16, b2, w3l, w3r, b3l, xw3_o, xr3_o):
    dinv = dinv16[...][:, :1]
    h2 = jnp.maximum(dinv * (s2[0] + s2[1] + t2[...]) + b2[...], 0.0)
    xw3_o[...] = _dot_t(h2, w3l[...])
    xr3_o[...] = _dot_t(h2, w3r[...]) + b3l[...]


def _dense3_body(s3, xr3, cntc16, dinv16, w4, t4_o):
    h3 = jnp.maximum((s3[0] + s3[1]) / cntc16[...][:, :1] + xr3[...], 0.0)
    t4_o[...] = dinv16[...][:, :1] * _dot_t(h3, w4[...])


def _dense4_body(s4, t4, dinv16, b4, wfc, bfc, out_o):
    h4 = jnp.maximum(
        dinv16[...][:, :1] * (s4[0] + s4[1] + t4[...]) + b4[...], 0.0)
    out_o[...] = _dot_t(h4, wfc[...]) + bfc[...]


def _o(minor):
    return jax.ShapeDtypeStruct((N_PAD, minor), jnp.float32)


def kernel(x, edge_index, W1l, b1l, W1r, W2, b2, W3l, b3l, W3r, W4, b4,
           Wfc, bfc):
    n = x.shape[0]
    e = edge_index.shape[1]
    d_in = x.shape[1]
    h1d = W1l.shape[0]
    h2d = W3l.shape[0]
    d_out = Wfc.shape[0]

    n_blk = -(-e // (NW * 128))
    e_pad = NW * 128 * n_blk
    src = edge_index[0].astype(jnp.int32)
    dst = edge_index[1].astype(jnp.int32)
    src3 = jnp.concatenate(
        [src, jnp.zeros((e_pad - e,), jnp.int32)]).reshape(NW, n_blk, 128)
    dst3 = jnp.concatenate(
        [dst, jnp.full((e_pad - e,), n, jnp.int32)]).reshape(NW, n_blk, 128)

    x_pad = jnp.pad(x, ((0, N_PAD - n), (0, 0)))
    zeros_w = jnp.zeros((ROWS_PT, d_in), jnp.float32)
    zeros_n = jnp.zeros((ROWS_PT, h2d), jnp.float32)
    zeros16 = jnp.zeros((ROWS_PT, 16), jnp.float32)
    ones16 = jnp.ones((128, 16), jnp.float32)

    spmm_w_cnt = _make_spmm(d_in, n_blk, True)
    spmm_w = _make_spmm(h1d, n_blk, False)
    spmm_n = _make_spmm(h2d, n_blk, False)

    b1l_ = b1l.reshape(1, -1)
    b2_ = b2.reshape(1, -1)
    b3l_ = b3l.reshape(1, -1)
    b4_ = b4.reshape(1, -1)
    bfc_ = bfc.reshape(1, -1)

    s1, c1 = spmm_w_cnt(x_pad, src3, dst3, zeros_w, zeros16, ones16)

    t2, dinv16, cntc16 = pl.pallas_call(
        _dense1_body,
        grid=(_GRID,),
        in_specs=[_pair(d_in), _pair(16), _rows(d_in), _full((h1d, d_in)),
                  _full((1, h1d)), _full((h1d, d_in)), _full((h1d, h1d))],
        out_specs=[_rows(h1d), _rows(16), _rows(16)],
        out_shape=[_o(h1d), _o(16), _o(16)],
    )(s1, c1, x_pad, W1l, b1l_, W1r, W2)

    (s2,) = spmm_w(t2, src3, dst3, zeros_w)

    xw3, xr3 = pl.pallas_call(
        _dense2_body,
        grid=(_GRID,),
        in_specs=[_pair(h1d), _rows(h1d), _rows(16), _full((1, h1d)),
                  _full((h2d, h1d)), _full((h2d, h1d)), _full((1, h2d))],
        out_specs=[_rows(h2d), _rows(h2d)],
        out_shape=[_o(h2d), _o(h2d)],
    )(s2, t2, dinv16, b2_, W3l, W3r, b3l_)

    (s3,) = spmm_n(xw3, src3, dst3, zeros_n)

    (t4,) = pl.pallas_call(
        _dense3_body,
        grid=(_GRID,),
        in_specs=[_pair(h2d), _rows(h2d), _rows(16), _rows(16),
                  _full((h2d, h2d))],
        out_specs=[_rows(h2d)],
        out_shape=[_o(h2d)],
    )(s3, xr3, cntc16, dinv16, W4)

    (s4,) = spmm_n(t4, src3, dst3, zeros_n)

    (out,) = pl.pallas_call(
        _dense4_body,
        grid=(_GRID,),
        in_specs=[_pair(h2d), _rows(h2d), _rows(16), _full((1, h2d)),
                  _full((d_out, h2d)), _full((1, d_out))],
        out_specs=[_rows(d_out)],
        out_shape=[_o(d_out)],
    )(s4, t4, dinv16, b4_, Wfc, bfc_)

    return out[:n]


# speedup vs baseline: 5.6549x; 5.6549x over previous
import functools

import jax
import jax.numpy as jnp
from jax import lax
from jax.experimental import pallas as pl
from jax.experimental.pallas import tpu as pltpu
from jax.experimental.pallas import tpu_sc as plsc

N_PAD = 10240
ROWS_PT = N_PAD // 16
NW = 32
_G = 8


def _make_spmm(d, n_blk):
    mesh = plsc.VectorSubcoreMesh(core_axis_name="c", subcore_axis_name="s")

    out_type = [jax.ShapeDtypeStruct((2, N_PAD, d), jnp.float32)]
    scratch = [
        pltpu.VMEM((_G, 128), jnp.int32),
        pltpu.VMEM((_G, 128), jnp.int32),
        pltpu.VMEM((128, d), jnp.float32),
        pltpu.VMEM_SHARED((N_PAD, d), jnp.float32),
        pltpu.SemaphoreType.DMA,
    ]

    def body(x_hbm, src_hbm, dst_hbm, zeros_hbm, out_hbm,
             sidx, didx, rows, acc, sem):
        c = lax.axis_index("c")
        s = lax.axis_index("s")
        w = c * 16 + s
        r0 = s * ROWS_PT

        pltpu.sync_copy(zeros_hbm, acc.at[pl.ds(r0, ROWS_PT)])
        plsc.subcore_barrier()

        n_grp = n_blk // _G

        def group(g, carry):
            pltpu.sync_copy(src_hbm.at[w * n_grp + g], sidx)
            pltpu.sync_copy(dst_hbm.at[w * n_grp + g], didx)
            for j in range(_G):
                pltpu.async_copy(x_hbm.at[sidx.at[j]], rows, sem).wait()
                pltpu.sync_copy(rows, acc.at[didx.at[j]], add=True)
            return carry

        lax.fori_loop(0, n_grp, group, 0)

        plsc.subcore_barrier()
        pltpu.sync_copy(acc.at[pl.ds(r0, ROWS_PT)],
                        out_hbm.at[c, pl.ds(r0, ROWS_PT)])

    return pl.kernel(body, out_type=out_type, mesh=mesh,
                     scratch_types=scratch)


def _make_cnt(n_blk):
    mesh = plsc.VectorSubcoreMesh(core_axis_name="c", subcore_axis_name="s")

    out_type = [jax.ShapeDtypeStruct((2, N_PAD, 128), jnp.float32)]
    scratch = [
        pltpu.VMEM((_G, 128), jnp.int32),
        pltpu.VMEM((128, 128), jnp.float32),
        pltpu.VMEM_SHARED((N_PAD, 128), jnp.float32),
    ]

    def body(dst_hbm, zeros_hbm, ones_hbm, out_hbm, didx, ones_v, acc):
        c = lax.axis_index("c")
        s = lax.axis_index("s")
        w = c * 16 + s
        r0 = s * ROWS_PT

        pltpu.sync_copy(zeros_hbm, acc.at[pl.ds(r0, ROWS_PT)])
        pltpu.sync_copy(ones_hbm, ones_v)
        plsc.subcore_barrier()

        n_grp = n_blk // _G

        def group(g, carry):
            pltpu.sync_copy(dst_hbm.at[w * n_grp + g], didx)
            for j in range(_G):
                pltpu.sync_copy(ones_v, acc.at[didx.at[j]], add=True)
            return carry

        lax.fori_loop(0, n_grp, group, 0)

        plsc.subcore_barrier()
        pltpu.sync_copy(acc.at[pl.ds(r0, ROWS_PT)],
                        out_hbm.at[c, pl.ds(r0, ROWS_PT)])

    return pl.kernel(body, out_type=out_type, mesh=mesh,
                     scratch_types=scratch)


def _dot_t(a, w):
    return lax.dot_general(a, w, (((1,), (1,)), ((), ())),
                           preferred_element_type=jnp.float32)


_R = 256
_GRID = N_PAD // _R


def _full(shape):
    return pl.BlockSpec(shape, lambda i: (0,) * len(shape))


def _rows(minor):
    return pl.BlockSpec((_R, minor), lambda i: (i, 0))


def _pair(minor):
    return pl.BlockSpec((2, _R, minor), lambda i: (0, i, 0))


def _dense1_body(s1, c1, x, w1l, b1l, w1r, w2, t2_o, dinv_o, cntc_o):
    cnt = c1[0][:, :1] + c1[1][:, :1]
    cntc = jnp.maximum(cnt, 1.0)
    agg = (s1[0] + s1[1]) / cntc
    h1 = jnp.maximum(
        _dot_t(agg, w1l[...]) + b1l[...] + _dot_t(x[...], w1r[...]), 0.0)
    dinv = lax.rsqrt(cnt + 1.0)
    t2_o[...] = dinv * _dot_t(h1, w2[...])
    dinv_o[...] = jnp.broadcast_to(dinv, (_R, 16))
    cntc_o[...] = jnp.broadcast_to(cntc, (_R, 16))


def _dense2_body(s2, t2, dinv16, b2, w3l, w3r, b3l, xcat_o):
    dinv = dinv16[...][:, :1]
    h2 = jnp.maximum(dinv * (s2[0] + s2[1] + t2[...]) + b2[...], 0.0)
    xcat_o[...] = jnp.concatenate(
        [_dot_t(h2, w3l[...]), _dot_t(h2, w3r[...]) + b3l[...]], axis=1)


def _dense3_body(s3, xcat, cntc16, dinv16, w4, t4_o):
    h = s3[0] + s3[1]
    h3 = jnp.maximum(
        h[:, :64] / cntc16[...][:, :1] + xcat[...][:, 64:], 0.0)
    t4 = dinv16[...][:, :1] * _dot_t(h3, w4[...])
    t4_o[...] = jnp.concatenate(
        [t4, jnp.zeros((_R, 64), jnp.float32)], axis=1)


def _dense4_body(s4, t4, dinv16, b4, wfc, bfc, out_o):
    h4 = jnp.maximum(
        dinv16[...][:, :1] * (s4[0][:, :64] + s4[1][:, :64] + t4[...][:, :64])
        + b4[...], 0.0)
    out_o[...] = _dot_t(h4, wfc[...]) + bfc[...]


def _o(minor):
    return jax.ShapeDtypeStruct((N_PAD, minor), jnp.float32)


def kernel(x, edge_index, W1l, b1l, W1r, W2, b2, W3l, b3l, W3r, W4, b4,
           Wfc, bfc):
    n = x.shape[0]
    e = edge_index.shape[1]
    d_in = x.shape[1]
    h1d = W1l.shape[0]
    h2d = W3l.shape[0]
    d_out = Wfc.shape[0]

    n_blk = _G * (-(-e // (NW * 128 * _G)))
    e_pad = NW * 128 * n_blk
    src = edge_index[0].astype(jnp.int32)
    dst = edge_index[1].astype(jnp.int32)
    n_grp = n_blk // _G
    src3 = jnp.concatenate(
        [src, jnp.zeros((e_pad - e,), jnp.int32)]).reshape(
            NW * n_grp, _G, 128)
    dst3 = jnp.concatenate(
        [dst, jnp.full((e_pad - e,), n, jnp.int32)]).reshape(
            NW * n_grp, _G, 128)

    x_pad = jnp.pad(x, ((0, N_PAD - n), (0, 0)))
    zeros_w = jnp.zeros((ROWS_PT, 128), jnp.float32)
    ones128 = jnp.ones((128, 128), jnp.float32)

    spmm_w = _make_spmm(128, n_blk)
    cnt_fn = _make_cnt(n_blk)

    b1l_ = b1l.reshape(1, -1)
    b2_ = b2.reshape(1, -1)
    b3l_ = b3l.reshape(1, -1)
    b4_ = b4.reshape(1, -1)
    bfc_ = bfc.reshape(1, -1)

    (s1,) = spmm_w(x_pad, src3, dst3, zeros_w)
    (c1,) = cnt_fn(dst3, zeros_w, ones128)

    t2, dinv16, cntc16 = pl.pallas_call(
        _dense1_body,
        grid=(_GRID,),
        in_specs=[_pair(d_in), _pair(128), _rows(d_in), _full((h1d, d_in)),
                  _full((1, h1d)), _full((h1d, d_in)), _full((h1d, h1d))],
        out_specs=[_rows(h1d), _rows(16), _rows(16)],
        out_shape=[_o(h1d), _o(16), _o(16)],
    )(s1, c1, x_pad, W1l, b1l_, W1r, W2)

    (s2,) = spmm_w(t2, src3, dst3, zeros_w)

    (xcat,) = pl.pallas_call(
        _dense2_body,
        grid=(_GRID,),
        in_specs=[_pair(h1d), _rows(h1d), _rows(16), _full((1, h1d)),
                  _full((h2d, h1d)), _full((h2d, h1d)), _full((1, h2d))],
        out_specs=[_rows(128)],
        out_shape=[_o(128)],
    )(s2, t2, dinv16, b2_, W3l, W3r, b3l_)

    (s3,) = spmm_w(xcat, src3, dst3, zeros_w)

    (t4,) = pl.pallas_call(
        _dense3_body,
        grid=(_GRID,),
        in_specs=[_pair(128), _rows(128), _rows(16), _rows(16),
                  _full((h2d, h2d))],
        out_specs=[_rows(128)],
        out_shape=[_o(128)],
    )(s3, xcat, cntc16, dinv16, W4)

    (s4,) = spmm_w(t4, src3, dst3, zeros_w)

    (out,) = pl.pallas_call(
        _dense4_body,
        grid=(_GRID,),
        in_specs=[_pair(128), _rows(128), _rows(16), _full((1, h2d)),
                  _full((d_out, h2d)), _full((1, d_out))],
        out_specs=[_rows(d_out)],
        out_shape=[_o(d_out)],
    )(s4, t4, dinv16, b4_, Wfc, bfc_)

    return out[:n]
